# Initial kernel scaffold; baseline (speedup 1.0000x reference)
import jax
import jax.numpy as jnp
from jax.experimental import pallas as pl


def kernel(central_idxs, context_idxs, negative_samples_idxs, central_weight, context_weight):
    raise NotImplementedError("write your pallas kernel here")


# speedup vs baseline: 5.2336x; 5.2336x over previous
import functools

import jax
import jax.numpy as jnp
from jax import lax
from jax.experimental import pallas as pl
from jax.experimental.pallas import tpu as pltpu
from jax.experimental.pallas import tpu_sc as plsc

L = 16

_GATHER_DN = lax.GatherDimensionNumbers(
    offset_dims=(), collapsed_slice_dims=(0,), start_index_map=(0,))


def _lanesum(v, lane):
    for sh in (8, 4, 2, 1):
        idx = lax.bitwise_xor(lane, sh)
        g = lax.gather(v, idx[:, None], dimension_numbers=_GATHER_DN,
                       slice_sizes=(1,),
                       mode=lax.GatherScatterMode.PROMISE_IN_BOUNDS)
        v = v + g
    return v


@functools.lru_cache(maxsize=None)
def _build_sc_kernel(B, K, D):
    info = plsc.get_sparse_core_info()
    NC, NS = info.num_cores, info.num_subcores
    NW = NC * NS
    BPW = B // NW
    C = 64
    NCHUNK = BPW // C
    CK = C * K
    IDXCH = 128
    NNEG = CK // IDXCH
    DB = D // L

    mesh = plsc.VectorSubcoreMesh(core_axis_name="c", subcore_axis_name="s")

    @functools.partial(
        pl.kernel,
        mesh=mesh,
        compiler_params=pltpu.CompilerParams(use_tc_tiling_on_sc=False),
        out_type=[
            jax.ShapeDtypeStruct((B,), jnp.float32),
            jax.ShapeDtypeStruct((B * K,), jnp.float32),
        ],
        scratch_types=[
            pltpu.VMEM((C,), jnp.int32),
            pltpu.VMEM((C,), jnp.int32),
            pltpu.VMEM((CK,), jnp.int32),
            pltpu.VMEM((C, D), jnp.float32),
            pltpu.VMEM((C, D), jnp.float32),
            pltpu.VMEM((CK, D), jnp.float32),
            pltpu.VMEM((C,), jnp.float32),
            pltpu.VMEM((CK,), jnp.float32),
            pltpu.SemaphoreType.DMA,
        ],
    )
    def k(cen_i_hbm, ctx_i_hbm, neg_i_hbm, cen_w, ctx_w,
          pos_hbm, negout_hbm,
          cen_i_v, ctx_i_v, neg_i_v, cen_r, ctx_r, neg_r, pos_v, neg_v, sem):
        wid = lax.axis_index("s") * NC + lax.axis_index("c")
        lane = lax.iota(jnp.int32, L)

        def chunk_body(ci, carry):
            base = wid * BPW + ci * C
            pltpu.sync_copy(cen_i_hbm.at[pl.ds(base, C)], cen_i_v)
            pltpu.sync_copy(ctx_i_hbm.at[pl.ds(base, C)], ctx_i_v)
            pltpu.sync_copy(neg_i_hbm.at[pl.ds(base * K, CK)], neg_i_v)
            cps = [
                pltpu.async_copy(cen_w.at[cen_i_v], cen_r, sem),
                pltpu.async_copy(ctx_w.at[ctx_i_v], ctx_r, sem),
            ]
            for j in range(NNEG):
                cps.append(pltpu.async_copy(
                    ctx_w.at[neg_i_v.at[pl.ds(j * IDXCH, IDXCH)]],
                    neg_r.at[pl.ds(j * IDXCH, IDXCH)], sem))
            for cp in cps:
                cp.wait()

            GR = 4
            NV = GR * K // L

            def g_body(bg, pos_vec):
                nvs = [jnp.zeros((L,), jnp.float32) for _ in range(NV)]
                for j in range(GR):
                    b = bg * GR + j
                    c = [cen_r[b, pl.ds(q * L, L)] for q in range(DB)]
                    t = [ctx_r[b, pl.ds(q * L, L)] for q in range(DB)]
                    acc = c[0] * t[0]
                    for q in range(1, DB):
                        acc = acc + c[q] * t[q]
                    ps = _lanesum(acc, lane)
                    pos_vec = jnp.where(lane == b % L, ps, pos_vec)
                    for kk in range(K):
                        n = [neg_r[b * K + kk, pl.ds(q * L, L)]
                             for q in range(DB)]
                        na = c[0] * n[0]
                        for q in range(1, DB):
                            na = na + c[q] * n[q]
                        s = _lanesum(na, lane)
                        p = j * K + kk
                        nvs[p // L] = jnp.where(lane == p % L, s, nvs[p // L])
                for v in range(NV):
                    neg_v[pl.ds(bg * (GR * K) + v * L, L)] = nvs[v]

                @pl.when(bg % GR == GR - 1)
                def _():
                    pos_v[pl.ds(bg * GR - (L - GR), L)] = pos_vec

                return pos_vec

            lax.fori_loop(0, C // GR, g_body, jnp.zeros((L,), jnp.float32))
            pltpu.sync_copy(pos_v, pos_hbm.at[pl.ds(base, C)])
            pltpu.sync_copy(neg_v.at[pl.ds(0, CK)],
                            negout_hbm.at[pl.ds(base * K, CK)])
            return carry

        lax.fori_loop(0, NCHUNK, chunk_body, jnp.int32(0))

    return k


def kernel(central_idxs, context_idxs, negative_samples_idxs,
           central_weight, context_weight):
    B, K = negative_samples_idxs.shape
    _, D = central_weight.shape
    cen_i = central_idxs.astype(jnp.int32)
    ctx_i = context_idxs.astype(jnp.int32)
    neg_i = negative_samples_idxs.astype(jnp.int32).reshape(B * K)
    f = _build_sc_kernel(B, K, D)
    pos, neg = f(cen_i, ctx_i, neg_i, central_weight, context_weight)
    return pos.reshape(B, 1), neg.reshape(B, K)
